# Initial kernel scaffold; baseline (speedup 1.0000x reference)
import jax
import jax.numpy as jnp
from jax.experimental import pallas as pl


def kernel(x, enc_W0, enc_b0, enc_W1, enc_b1, enc_W2, enc_b2, enc_W3, enc_b3, dec_W0, dec_b0, dec_W1, dec_b1, dec_W2, dec_b2, dec_W3, dec_b3, cb0, cb1, cb2, cb3):
    raise NotImplementedError("write your pallas kernel here")


# speedup vs baseline: 1.3848x; 1.3848x over previous
import jax
import jax.numpy as jnp
from jax.experimental import pallas as pl

_BATCH = 4096
_E = 32
_NC = 256
_NL = 4
_BETA = 0.25
_BLK = 512
_GRID = _BATCH // _BLK
_LOSS_SCALE = (1.0 + _BETA) / (_NL * _BATCH * _E)


def _rqvae_block(x_ref,
                 ew0, eb0, ew1, eb1, ew2, eb2, ew3, eb3,
                 dw0, db0, dw1, db1, dw2, db2, dw3, db3,
                 cb_ref, cbt_ref,
                 out_ref, loss_ref, idx_ref, oh_ref, lg_ref):
    hi = jax.lax.Precision.HIGHEST
    x = x_ref[...]
    h = jnp.maximum(jnp.dot(x, ew0[...]) + eb0[...], 0.0)
    h = jnp.maximum(jnp.dot(h, ew1[...]) + eb1[...], 0.0)
    h = jnp.maximum(jnp.dot(h, ew2[...]) + eb2[...], 0.0)
    latent = jnp.dot(h, ew3[...]) + eb3[...]

    residual = latent
    xq_acc = jnp.zeros_like(latent)
    sq_acc = jnp.float32(0.0)
    for l in range(_NL):
        cb = cb_ref[l]
        cbt = cbt_ref[l]
        d = (jnp.sum(residual * residual, axis=1, keepdims=True)
             + jnp.sum(cb * cb, axis=1)[None, :]
             - 2.0 * jnp.dot(residual, cbt))
        dmin = jnp.min(d, axis=1, keepdims=True)
        lane = jax.lax.broadcasted_iota(jnp.int32, d.shape, 1)
        idx = jnp.min(jnp.where(d == dmin, lane, _NC), axis=1)
        oh = (lane == idx[:, None]).astype(jnp.float32)
        xq = jnp.dot(oh, cb, precision=hi)
        diff = xq - residual
        sq_acc = sq_acc + jnp.sum(diff * diff)
        lg_ref[l] = d
        oh_ref[l] = oh
        idx_ref[l] = idx
        xq_st = residual + diff
        residual = residual - xq_st
        xq_acc = xq_acc + xq_st

    h = jnp.maximum(jnp.dot(xq_acc, dw0[...]) + db0[...], 0.0)
    h = jnp.maximum(jnp.dot(h, dw1[...]) + db1[...], 0.0)
    h = jnp.maximum(jnp.dot(h, dw2[...]) + db2[...], 0.0)
    out_ref[...] = jnp.dot(h, dw3[...]) + db3[...]

    i = pl.program_id(0)

    @pl.when(i == 0)
    def _():
        loss_ref[...] = jnp.zeros((1, 1), jnp.float32)

    loss_ref[...] += sq_acc.reshape(1, 1)

    @pl.when(i == _GRID - 1)
    def _():
        loss_ref[...] = loss_ref[...] * _LOSS_SCALE


def _const(shape):
    nd = len(shape)
    return pl.BlockSpec(shape, lambda i: (0,) * nd)


def kernel(x, enc_W0, enc_b0, enc_W1, enc_b1, enc_W2, enc_b2, enc_W3, enc_b3,
           dec_W0, dec_b0, dec_W1, dec_b1, dec_W2, dec_b2, dec_W3, dec_b3,
           cb0, cb1, cb2, cb3):
    ews = [enc_W0.T, enc_W1.T, enc_W2.T, enc_W3.T]
    ebs = [enc_b0[None, :], enc_b1[None, :], enc_b2[None, :], enc_b3[None, :]]
    dws = [dec_W0.T, dec_W1.T, dec_W2.T, dec_W3.T]
    dbs = [dec_b0[None, :], dec_b1[None, :], dec_b2[None, :], dec_b3[None, :]]
    cbs = jnp.stack([cb0, cb1, cb2, cb3], axis=0)
    cbts = jnp.stack([cb0.T, cb1.T, cb2.T, cb3.T], axis=0)

    in_specs = [pl.BlockSpec((_BLK, x.shape[1]), lambda i: (i, 0))]
    flat_w = []
    for w, b in zip(ews, ebs):
        flat_w += [w, b]
        in_specs += [_const(w.shape), _const(b.shape)]
    for w, b in zip(dws, dbs):
        flat_w += [w, b]
        in_specs += [_const(w.shape), _const(b.shape)]
    in_specs += [_const(cbs.shape), _const(cbts.shape)]

    out_shapes = (
        jax.ShapeDtypeStruct((_BATCH, x.shape[1]), jnp.float32),
        jax.ShapeDtypeStruct((1, 1), jnp.float32),
        jax.ShapeDtypeStruct((_NL, _BATCH), jnp.int32),
        jax.ShapeDtypeStruct((_NL, _BATCH, _NC), jnp.float32),
        jax.ShapeDtypeStruct((_NL, _BATCH, _NC), jnp.float32),
    )
    out_specs = (
        pl.BlockSpec((_BLK, x.shape[1]), lambda i: (i, 0)),
        pl.BlockSpec((1, 1), lambda i: (0, 0)),
        pl.BlockSpec((_NL, _BLK), lambda i: (0, i)),
        pl.BlockSpec((_NL, _BLK, _NC), lambda i: (0, i, 0)),
        pl.BlockSpec((_NL, _BLK, _NC), lambda i: (0, i, 0)),
    )

    out, loss, idx, oh, lg = pl.pallas_call(
        _rqvae_block,
        grid=(_GRID,),
        in_specs=in_specs,
        out_specs=out_specs,
        out_shape=out_shapes,
    )(x, *flat_w, cbs, cbts)

    return (out,
            loss.reshape(()),
            idx.T,
            jnp.transpose(oh, (1, 0, 2)),
            jnp.transpose(lg, (1, 0, 2)))


# speedup vs baseline: 2.0196x; 1.4584x over previous
import jax
import jax.numpy as jnp
from jax.experimental import pallas as pl

_BATCH = 4096
_E = 32
_NC = 256
_NL = 4
_BETA = 0.25
_BLK = 512
_GRID = _BATCH // _BLK
_LOSS_SCALE = (1.0 + _BETA) / (_NL * _BATCH * _E)


def _rqvae_block(x_ref,
                 ew0, eb0, ew1, eb1, ew2, eb2, ew3, eb3,
                 dw0, db0, dw1, db1, dw2, db2, dw3, db3,
                 cb_ref, cbt_ref,
                 out_ref, loss_ref, idx_ref, oh_ref, lg_ref):
    hi = jax.lax.Precision.HIGHEST
    x = x_ref[...]
    h = jnp.maximum(jnp.dot(x, ew0[...]) + eb0[...], 0.0)
    h = jnp.maximum(jnp.dot(h, ew1[...]) + eb1[...], 0.0)
    h = jnp.maximum(jnp.dot(h, ew2[...]) + eb2[...], 0.0)
    latent = jnp.dot(h, ew3[...]) + eb3[...]

    residual = latent
    xq_acc = jnp.zeros_like(latent)
    sq_acc = jnp.float32(0.0)
    for l in range(_NL):
        cb = cb_ref[l]
        cbt = cbt_ref[l]
        d = (jnp.sum(residual * residual, axis=1, keepdims=True)
             + jnp.sum(cb * cb, axis=1)[None, :]
             - 2.0 * jnp.dot(residual, cbt))
        dmin = jnp.min(d, axis=1, keepdims=True)
        lane = jax.lax.broadcasted_iota(jnp.int32, d.shape, 1)
        idx = jnp.min(jnp.where(d == dmin, lane, _NC), axis=1)
        oh = (lane == idx[:, None]).astype(jnp.float32)
        xq = jnp.dot(oh, cb, precision=hi)
        diff = xq - residual
        sq_acc = sq_acc + jnp.sum(diff * diff)
        lg_ref[:, l, :] = d
        oh_ref[:, l, :] = oh
        idx_ref[l] = idx
        xq_st = residual + diff
        residual = residual - xq_st
        xq_acc = xq_acc + xq_st

    h = jnp.maximum(jnp.dot(xq_acc, dw0[...]) + db0[...], 0.0)
    h = jnp.maximum(jnp.dot(h, dw1[...]) + db1[...], 0.0)
    h = jnp.maximum(jnp.dot(h, dw2[...]) + db2[...], 0.0)
    out_ref[...] = jnp.dot(h, dw3[...]) + db3[...]

    i = pl.program_id(0)

    @pl.when(i == 0)
    def _():
        loss_ref[...] = jnp.zeros((1, 1), jnp.float32)

    loss_ref[...] += sq_acc.reshape(1, 1)

    @pl.when(i == _GRID - 1)
    def _():
        loss_ref[...] = loss_ref[...] * _LOSS_SCALE


def _const(shape):
    nd = len(shape)
    return pl.BlockSpec(shape, lambda i: (0,) * nd)


def kernel(x, enc_W0, enc_b0, enc_W1, enc_b1, enc_W2, enc_b2, enc_W3, enc_b3,
           dec_W0, dec_b0, dec_W1, dec_b1, dec_W2, dec_b2, dec_W3, dec_b3,
           cb0, cb1, cb2, cb3):
    ews = [enc_W0.T, enc_W1.T, enc_W2.T, enc_W3.T]
    ebs = [enc_b0[None, :], enc_b1[None, :], enc_b2[None, :], enc_b3[None, :]]
    dws = [dec_W0.T, dec_W1.T, dec_W2.T, dec_W3.T]
    dbs = [dec_b0[None, :], dec_b1[None, :], dec_b2[None, :], dec_b3[None, :]]
    cbs = jnp.stack([cb0, cb1, cb2, cb3], axis=0)
    cbts = jnp.stack([cb0.T, cb1.T, cb2.T, cb3.T], axis=0)

    in_specs = [pl.BlockSpec((_BLK, x.shape[1]), lambda i: (i, 0))]
    flat_w = []
    for w, b in zip(ews, ebs):
        flat_w += [w, b]
        in_specs += [_const(w.shape), _const(b.shape)]
    for w, b in zip(dws, dbs):
        flat_w += [w, b]
        in_specs += [_const(w.shape), _const(b.shape)]
    in_specs += [_const(cbs.shape), _const(cbts.shape)]

    out_shapes = (
        jax.ShapeDtypeStruct((_BATCH, x.shape[1]), jnp.float32),
        jax.ShapeDtypeStruct((1, 1), jnp.float32),
        jax.ShapeDtypeStruct((_NL, _BATCH), jnp.int32),
        jax.ShapeDtypeStruct((_BATCH, _NL, _NC), jnp.float32),
        jax.ShapeDtypeStruct((_BATCH, _NL, _NC), jnp.float32),
    )
    out_specs = (
        pl.BlockSpec((_BLK, x.shape[1]), lambda i: (i, 0)),
        pl.BlockSpec((1, 1), lambda i: (0, 0)),
        pl.BlockSpec((_NL, _BLK), lambda i: (0, i)),
        pl.BlockSpec((_BLK, _NL, _NC), lambda i: (i, 0, 0)),
        pl.BlockSpec((_BLK, _NL, _NC), lambda i: (i, 0, 0)),
    )

    out, loss, idx, oh, lg = pl.pallas_call(
        _rqvae_block,
        grid=(_GRID,),
        in_specs=in_specs,
        out_specs=out_specs,
        out_shape=out_shapes,
    )(x, *flat_w, cbs, cbts)

    return (out, loss.reshape(()), idx.T, oh, lg)


# speedup vs baseline: 2.1058x; 1.0427x over previous
import jax
import jax.numpy as jnp
from jax.experimental import pallas as pl

_BATCH = 4096
_E = 32
_NC = 256
_NL = 4
_BETA = 0.25
_BLK = 1024
_GRID = _BATCH // _BLK
_LOSS_SCALE = (1.0 + _BETA) / (_NL * _BATCH * _E)


def _rqvae_block(x_ref,
                 ew0, eb0, ew1, eb1, ew2, eb2, ew3, eb3,
                 dw0, db0, dw1, db1, dw2, db2, dw3, db3,
                 cb_ref, cbt_ref,
                 out_ref, loss_ref, idx_ref, oh_ref, lg_ref):
    hi = jax.lax.Precision.HIGHEST
    x = x_ref[...]
    h = jnp.maximum(jnp.dot(x, ew0[...]) + eb0[...], 0.0)
    h = jnp.maximum(jnp.dot(h, ew1[...]) + eb1[...], 0.0)
    h = jnp.maximum(jnp.dot(h, ew2[...]) + eb2[...], 0.0)
    latent = jnp.dot(h, ew3[...]) + eb3[...]

    residual = latent
    xq_acc = jnp.zeros_like(latent)
    sq_acc = jnp.float32(0.0)
    for l in range(_NL):
        cb = cb_ref[l]
        cbt = cbt_ref[l]
        d = (jnp.sum(residual * residual, axis=1, keepdims=True)
             + jnp.sum(cb * cb, axis=1)[None, :]
             - 2.0 * jnp.dot(residual, cbt))
        dmin = jnp.min(d, axis=1, keepdims=True)
        lane = jax.lax.broadcasted_iota(jnp.int32, d.shape, 1)
        idx = jnp.min(jnp.where(d == dmin, lane, _NC), axis=1)
        oh = (lane == idx[:, None]).astype(jnp.float32)
        xq = jnp.dot(oh, cb, precision=hi)
        diff = xq - residual
        sq_acc = sq_acc + jnp.sum(diff * diff)
        lg_ref[:, l, :] = d
        oh_ref[:, l, :] = oh
        idx_ref[l] = idx
        xq_st = residual + diff
        residual = residual - xq_st
        xq_acc = xq_acc + xq_st

    h = jnp.maximum(jnp.dot(xq_acc, dw0[...]) + db0[...], 0.0)
    h = jnp.maximum(jnp.dot(h, dw1[...]) + db1[...], 0.0)
    h = jnp.maximum(jnp.dot(h, dw2[...]) + db2[...], 0.0)
    out_ref[...] = jnp.dot(h, dw3[...]) + db3[...]

    i = pl.program_id(0)

    @pl.when(i == 0)
    def _():
        loss_ref[...] = jnp.zeros((1, 1), jnp.float32)

    loss_ref[...] += sq_acc.reshape(1, 1)

    @pl.when(i == _GRID - 1)
    def _():
        loss_ref[...] = loss_ref[...] * _LOSS_SCALE


def _const(shape):
    nd = len(shape)
    return pl.BlockSpec(shape, lambda i: (0,) * nd)


def kernel(x, enc_W0, enc_b0, enc_W1, enc_b1, enc_W2, enc_b2, enc_W3, enc_b3,
           dec_W0, dec_b0, dec_W1, dec_b1, dec_W2, dec_b2, dec_W3, dec_b3,
           cb0, cb1, cb2, cb3):
    ews = [enc_W0.T, enc_W1.T, enc_W2.T, enc_W3.T]
    ebs = [enc_b0[None, :], enc_b1[None, :], enc_b2[None, :], enc_b3[None, :]]
    dws = [dec_W0.T, dec_W1.T, dec_W2.T, dec_W3.T]
    dbs = [dec_b0[None, :], dec_b1[None, :], dec_b2[None, :], dec_b3[None, :]]
    cbs = jnp.stack([cb0, cb1, cb2, cb3], axis=0)
    cbts = jnp.stack([cb0.T, cb1.T, cb2.T, cb3.T], axis=0)

    in_specs = [pl.BlockSpec((_BLK, x.shape[1]), lambda i: (i, 0))]
    flat_w = []
    for w, b in zip(ews, ebs):
        flat_w += [w, b]
        in_specs += [_const(w.shape), _const(b.shape)]
    for w, b in zip(dws, dbs):
        flat_w += [w, b]
        in_specs += [_const(w.shape), _const(b.shape)]
    in_specs += [_const(cbs.shape), _const(cbts.shape)]

    out_shapes = (
        jax.ShapeDtypeStruct((_BATCH, x.shape[1]), jnp.float32),
        jax.ShapeDtypeStruct((1, 1), jnp.float32),
        jax.ShapeDtypeStruct((_NL, _BATCH), jnp.int32),
        jax.ShapeDtypeStruct((_BATCH, _NL, _NC), jnp.float32),
        jax.ShapeDtypeStruct((_BATCH, _NL, _NC), jnp.float32),
    )
    out_specs = (
        pl.BlockSpec((_BLK, x.shape[1]), lambda i: (i, 0)),
        pl.BlockSpec((1, 1), lambda i: (0, 0)),
        pl.BlockSpec((_NL, _BLK), lambda i: (0, i)),
        pl.BlockSpec((_BLK, _NL, _NC), lambda i: (i, 0, 0)),
        pl.BlockSpec((_BLK, _NL, _NC), lambda i: (i, 0, 0)),
    )

    out, loss, idx, oh, lg = pl.pallas_call(
        _rqvae_block,
        grid=(_GRID,),
        in_specs=in_specs,
        out_specs=out_specs,
        out_shape=out_shapes,
    )(x, *flat_w, cbs, cbts)

    return (out, loss.reshape(()), idx.T, oh, lg)


# speedup vs baseline: 2.3762x; 1.1284x over previous
import jax
import jax.numpy as jnp
from jax.experimental import pallas as pl

_BATCH = 4096
_E = 32
_NC = 256
_NL = 4
_BETA = 0.25
_BLK = 1024
_GRID = _BATCH // _BLK
_LOSS_SCALE = (1.0 + _BETA) / (_NL * _BATCH * _E)


def _rqvae_block(x_ref,
                 ew0, eb0, ew1, eb1, ew2, eb2, ew3, eb3,
                 dw0, db0, dw1, db1, dw2, db2, dw3, db3,
                 cb_ref, cbt_ref, cbh_ref, cbm_ref, cbl_ref,
                 out_ref, loss_ref, idx_ref, oh_ref, lg_ref):
    x = x_ref[...]
    h = jnp.maximum(jnp.dot(x, ew0[...]) + eb0[...], 0.0)
    h = jnp.maximum(jnp.dot(h, ew1[...]) + eb1[...], 0.0)
    h = jnp.maximum(jnp.dot(h, ew2[...]) + eb2[...], 0.0)
    latent = jnp.dot(h, ew3[...]) + eb3[...]

    residual = latent
    xq_acc = jnp.zeros_like(latent)
    sq_acc = jnp.float32(0.0)
    for l in range(_NL):
        cb = cb_ref[l]
        cbt = cbt_ref[l]
        d = (jnp.sum(residual * residual, axis=1, keepdims=True)
             + jnp.sum(cb * cb, axis=1)[None, :]
             - 2.0 * jnp.dot(residual, cbt))
        dmin = jnp.min(d, axis=1, keepdims=True)
        lane = jax.lax.broadcasted_iota(jnp.int32, d.shape, 1)
        idx = jnp.min(jnp.where(d == dmin, lane, _NC), axis=1)
        oh = (lane == idx[:, None]).astype(jnp.float32)
        xq = ((jnp.dot(oh, cbh_ref[l]) + jnp.dot(oh, cbm_ref[l]))
              + jnp.dot(oh, cbl_ref[l]))
        diff = xq - residual
        sq_acc = sq_acc + jnp.sum(diff * diff)
        lg_ref[:, l, :] = d
        oh_ref[:, l, :] = oh
        idx_ref[l] = idx
        xq_st = residual + diff
        residual = residual - xq_st
        xq_acc = xq_acc + xq_st

    h = jnp.maximum(jnp.dot(xq_acc, dw0[...]) + db0[...], 0.0)
    h = jnp.maximum(jnp.dot(h, dw1[...]) + db1[...], 0.0)
    h = jnp.maximum(jnp.dot(h, dw2[...]) + db2[...], 0.0)
    out_ref[...] = jnp.dot(h, dw3[...]) + db3[...]

    i = pl.program_id(0)

    @pl.when(i == 0)
    def _():
        loss_ref[...] = jnp.zeros((1, 1), jnp.float32)

    loss_ref[...] += sq_acc.reshape(1, 1)

    @pl.when(i == _GRID - 1)
    def _():
        loss_ref[...] = loss_ref[...] * _LOSS_SCALE


def _const(shape):
    nd = len(shape)
    return pl.BlockSpec(shape, lambda i: (0,) * nd)


def kernel(x, enc_W0, enc_b0, enc_W1, enc_b1, enc_W2, enc_b2, enc_W3, enc_b3,
           dec_W0, dec_b0, dec_W1, dec_b1, dec_W2, dec_b2, dec_W3, dec_b3,
           cb0, cb1, cb2, cb3):
    ews = [enc_W0.T, enc_W1.T, enc_W2.T, enc_W3.T]
    ebs = [enc_b0[None, :], enc_b1[None, :], enc_b2[None, :], enc_b3[None, :]]
    dws = [dec_W0.T, dec_W1.T, dec_W2.T, dec_W3.T]
    dbs = [dec_b0[None, :], dec_b1[None, :], dec_b2[None, :], dec_b3[None, :]]
    cbs = jnp.stack([cb0, cb1, cb2, cb3], axis=0)
    cbts = jnp.stack([cb0.T, cb1.T, cb2.T, cb3.T], axis=0)
    cb_h = cbs.astype(jnp.bfloat16).astype(jnp.float32)
    _r1 = cbs - cb_h
    cb_m = _r1.astype(jnp.bfloat16).astype(jnp.float32)
    cb_l = _r1 - cb_m

    in_specs = [pl.BlockSpec((_BLK, x.shape[1]), lambda i: (i, 0))]
    flat_w = []
    for w, b in zip(ews, ebs):
        flat_w += [w, b]
        in_specs += [_const(w.shape), _const(b.shape)]
    for w, b in zip(dws, dbs):
        flat_w += [w, b]
        in_specs += [_const(w.shape), _const(b.shape)]
    in_specs += [_const(cbs.shape), _const(cbts.shape),
                 _const(cbs.shape), _const(cbs.shape), _const(cbs.shape)]

    out_shapes = (
        jax.ShapeDtypeStruct((_BATCH, x.shape[1]), jnp.float32),
        jax.ShapeDtypeStruct((1, 1), jnp.float32),
        jax.ShapeDtypeStruct((_NL, _BATCH), jnp.int32),
        jax.ShapeDtypeStruct((_BATCH, _NL, _NC), jnp.float32),
        jax.ShapeDtypeStruct((_BATCH, _NL, _NC), jnp.float32),
    )
    out_specs = (
        pl.BlockSpec((_BLK, x.shape[1]), lambda i: (i, 0)),
        pl.BlockSpec((1, 1), lambda i: (0, 0)),
        pl.BlockSpec((_NL, _BLK), lambda i: (0, i)),
        pl.BlockSpec((_BLK, _NL, _NC), lambda i: (i, 0, 0)),
        pl.BlockSpec((_BLK, _NL, _NC), lambda i: (i, 0, 0)),
    )

    out, loss, idx, oh, lg = pl.pallas_call(
        _rqvae_block,
        grid=(_GRID,),
        in_specs=in_specs,
        out_specs=out_specs,
        out_shape=out_shapes,
    )(x, *flat_w, cbs, cbts, cb_h, cb_m, cb_l)

    return (out, loss.reshape(()), idx.T, oh, lg)
